# baseline (speedup 1.0000x reference)
import jax
import jax.numpy as jnp
from jax import lax
from jax.experimental import pallas as pl
from jax.experimental.pallas import tpu as pltpu
from jax.experimental.pallas import tpu_sc as plsc

N_NODES = 50000
N_EDGES = 800000
PAYLOAD = 32

NUM_SUBCORES = 16
EDGES_PER_TILE = N_EDGES // NUM_SUBCORES
CHUNK = 400
CHUNKS_PER_TILE = EDGES_PER_TILE // CHUNK
GROUPS = CHUNK // 16
SCATTER_ROWS = 100
SCATTERS = CHUNK // SCATTER_ROWS
N_A = 50048
ROWS_PER_TILE = N_A // NUM_SUBCORES
ZCOPIES = ROWS_PER_TILE // CHUNK


def _sums_body(x_ref, o_ref):
    o_ref[...] = jnp.sum(x_ref[...], axis=1, keepdims=True)


def _node_sums(node_feats):
    n = node_feats.shape[0]
    out = pl.pallas_call(
        _sums_body,
        grid=(50,),
        in_specs=[pl.BlockSpec((n // 50, 128), lambda i: (i, 0))],
        out_specs=pl.BlockSpec((n // 50, 1), lambda i: (i, 0)),
        out_shape=jax.ShapeDtypeStruct((n, 1), jnp.float32),
    )(node_feats)
    return out.reshape(n)


def _sc_body(sums_hbm, snd_hbm, rcv_hbm, radial_hbm, a_hbm, comb_hbm,
             buf0, buf1, snd0, snd1, rcv0, rcv1, s_v, sums_sh, a_sh,
             lsem0, lsem1, dsem0, dsem1):
    s = lax.axis_index("s")
    zeros16 = jnp.zeros((16,), jnp.float32)
    lanes = lax.iota(jnp.int32, 16)
    e_base = s * EDGES_PER_TILE
    rz = s * ROWS_PER_TILE
    bufs, snds, rcvs = (buf0, buf1), (snd0, snd1), (rcv0, rcv1)
    lsems, dsems = (lsem0, lsem1), (dsem0, dsem1)

    def zrow(i, carry):
        buf0[i, pl.ds(0, 16)] = zeros16
        buf0[i, pl.ds(16, 16)] = zeros16
        return carry

    lax.fori_loop(0, CHUNK, zrow, 0)
    for i in range(ZCOPIES):
        pltpu.sync_copy(buf0.at[pl.ds(0, CHUNK)],
                        a_sh.at[pl.ds(rz + i * CHUNK, CHUNK)])
    rest = ROWS_PER_TILE - ZCOPIES * CHUNK
    pltpu.sync_copy(buf0.at[pl.ds(0, rest)],
                    a_sh.at[pl.ds(rz + ZCOPIES * CHUNK, rest)])

    @pl.when(s == 0)
    def _():
        pltpu.sync_copy(sums_hbm, sums_sh)

    plsc.subcore_barrier()

    def issue_loads(k, par):
        e0 = e_base + k * CHUNK
        pltpu.async_copy(radial_hbm.at[pl.ds(e0, CHUNK)], bufs[par],
                         lsems[par])
        pltpu.async_copy(snd_hbm.at[pl.ds(e0, CHUNK)], snds[par], lsems[par])
        pltpu.async_copy(rcv_hbm.at[e0 // CHUNK], rcvs[par], lsems[par])

    def wait_loads(k, par):
        e0 = e_base + k * CHUNK
        pltpu.make_async_copy(radial_hbm.at[pl.ds(e0, CHUNK)], bufs[par],
                              lsems[par]).wait()
        pltpu.make_async_copy(snd_hbm.at[pl.ds(e0, CHUNK)], snds[par],
                              lsems[par]).wait()
        pltpu.make_async_copy(rcv_hbm.at[e0 // CHUNK], rcvs[par],
                              lsems[par]).wait()

    def wait_drain(par):
        pltpu.make_async_copy(bufs[par], comb_hbm.at[pl.ds(0, CHUNK)],
                              dsems[par]).wait()

    def process(k, par, issue_next, drain_guard):
        other = 1 - par
        if issue_next:
            if drain_guard is None:
                wait_drain(other)
            else:
                @pl.when(drain_guard)
                def _():
                    wait_drain(other)
            issue_loads(k + 1, other)
        wait_loads(k, par)
        pltpu.sync_copy(sums_sh.at[snds[par]], s_v)
        buf = bufs[par]

        def grp(g, gc):
            b = g * 16
            s16 = s_v[pl.ds(b, 16)]
            rows = lanes + b
            for j in range(PAYLOAD):
                cols = jnp.full((16,), j, jnp.int32)
                v = plsc.load_gather(buf, [rows, cols])
                plsc.store_scatter(buf, [rows, cols], v * s16)
            return gc

        lax.fori_loop(0, GROUPS, grp, 0)
        e0 = e_base + k * CHUNK
        pltpu.async_copy(buf, comb_hbm.at[pl.ds(e0, CHUNK)], dsems[par])
        for j in range(SCATTERS):
            pltpu.sync_copy(buf.at[pl.ds(j * SCATTER_ROWS, SCATTER_ROWS)],
                            a_sh.at[rcvs[par].at[j]], add=True)

    issue_loads(0, 0)

    def outer(i, carry):
        k0 = 2 * i
        process(k0, 0, True, i > 0)
        process(k0 + 1, 1, True, None)
        return carry

    lax.fori_loop(0, (CHUNKS_PER_TILE - 1) // 2, outer, 0)
    process(CHUNKS_PER_TILE - 1, 0, False, None)
    wait_drain(0)
    wait_drain(1)
    plsc.subcore_barrier()
    pltpu.sync_copy(a_sh.at[pl.ds(rz, ROWS_PER_TILE)],
                    a_hbm.at[pl.ds(rz, ROWS_PER_TILE)])


def _sc_channel(sums, sender, rcv3d, radial_c):
    run = pl.kernel(
        _sc_body,
        out_type=[
            jax.ShapeDtypeStruct((N_A, PAYLOAD), jnp.float32),
            jax.ShapeDtypeStruct((N_EDGES, PAYLOAD), jnp.float32),
        ],
        mesh=plsc.VectorSubcoreMesh(core_axis_name="c", subcore_axis_name="s",
                                    num_cores=1),
        scratch_types=[
            pltpu.VMEM((CHUNK, PAYLOAD), jnp.float32),
            pltpu.VMEM((CHUNK, PAYLOAD), jnp.float32),
            pltpu.VMEM((CHUNK,), jnp.int32),
            pltpu.VMEM((CHUNK,), jnp.int32),
            pltpu.VMEM((SCATTERS, SCATTER_ROWS), jnp.int32),
            pltpu.VMEM((SCATTERS, SCATTER_ROWS), jnp.int32),
            pltpu.VMEM((CHUNK,), jnp.float32),
            pltpu.VMEM_SHARED((N_NODES,), jnp.float32),
            pltpu.VMEM_SHARED((N_A, PAYLOAD), jnp.float32),
            pltpu.SemaphoreType.DMA,
            pltpu.SemaphoreType.DMA,
            pltpu.SemaphoreType.DMA,
            pltpu.SemaphoreType.DMA,
        ],
        compiler_params=pltpu.CompilerParams(
            needs_layout_passes=False, use_tc_tiling_on_sc=False),
    )
    return run(sums, sender, rcv3d, radial_c)


def kernel(edge_index, radial_feature, node_feats):
    e = radial_feature.shape[1]
    k, i, j = radial_feature.shape[2:]
    n = node_feats.shape[0]
    sender = edge_index[0].astype(jnp.int32)
    receiver = edge_index[1].astype(jnp.int32)
    radial = radial_feature.reshape(2, e, k * i * j)
    sums = _node_sums(node_feats)
    rcv3d = receiver.reshape(e // CHUNK, SCATTERS, SCATTER_ROWS)
    a_r, comb_r = _sc_channel(sums, sender, rcv3d, radial[0])
    a_i, comb_i = _sc_channel(sums, sender, rcv3d, radial[1])
    return (
        a_r[:n].reshape(n, k, i, j),
        a_i[:n].reshape(n, k, i, j),
        comb_r.reshape(e, k, i, j),
        comb_i.reshape(e, k, i, j),
    )


# speedup vs baseline: 1.1757x; 1.1757x over previous
import jax
import jax.numpy as jnp
from jax import lax
from jax.experimental import pallas as pl
from jax.experimental.pallas import tpu as pltpu
from jax.experimental.pallas import tpu_sc as plsc

N_NODES = 50000
N_EDGES = 800000
PAYLOAD = 32

NUM_SUBCORES = 16
EDGES_PER_TILE = N_EDGES // NUM_SUBCORES
CHUNK = 400
CHUNKS_PER_TILE = EDGES_PER_TILE // CHUNK
GROUPS = CHUNK // 16
SCATTER_ROWS = 100
SCATTERS = CHUNK // SCATTER_ROWS
N_A = 50048
ROWS_PER_TILE = N_A // NUM_SUBCORES
ZCOPIES = ROWS_PER_TILE // CHUNK


def _sums_body(x_ref, o_ref):
    o_ref[...] = jnp.sum(x_ref[...], axis=1, keepdims=True)


def _node_sums(node_feats):
    n = node_feats.shape[0]
    out = pl.pallas_call(
        _sums_body,
        grid=(50,),
        in_specs=[pl.BlockSpec((n // 50, 128), lambda i: (i, 0))],
        out_specs=pl.BlockSpec((n // 50, 1), lambda i: (i, 0)),
        out_shape=jax.ShapeDtypeStruct((n, 1), jnp.float32),
    )(node_feats)
    return out.reshape(n)


def _sc_body(sums_hbm, snd_hbm, rcv_hbm, radial_hbm, a_hbm, comb_hbm,
             buf0, buf1, snd0, snd1, rcv_v, s0, s1, sums_sh, a_sh,
             lsem0, lsem1, dsem0, dsem1, ssem0, ssem1,
             nsem0, nsem1, rsem0, rsem1):
    s = lax.axis_index("s")
    c = lax.axis_index("c")
    zeros16 = jnp.zeros((16,), jnp.float32)
    lanes = lax.iota(jnp.int32, 16)
    e_base = s * EDGES_PER_TILE
    rz = s * ROWS_PER_TILE
    rad_hbm = radial_hbm.at[c]
    ach_hbm = a_hbm.at[c]
    cch_hbm = comb_hbm.at[c]
    bufs, snds, svs = (buf0, buf1), (snd0, snd1), (s0, s1)
    lsems, dsems, ssems = (lsem0, lsem1), (dsem0, dsem1), (ssem0, ssem1)
    nsems, rsems = (nsem0, nsem1), (rsem0, rsem1)

    def zrow(i, carry):
        buf0[i, pl.ds(0, 16)] = zeros16
        buf0[i, pl.ds(16, 16)] = zeros16
        return carry

    lax.fori_loop(0, CHUNK, zrow, 0)
    for i in range(ZCOPIES):
        pltpu.sync_copy(buf0.at[pl.ds(0, CHUNK)],
                        a_sh.at[pl.ds(rz + i * CHUNK, CHUNK)])
    rest = ROWS_PER_TILE - ZCOPIES * CHUNK
    pltpu.sync_copy(buf0.at[pl.ds(0, rest)],
                    a_sh.at[pl.ds(rz + ZCOPIES * CHUNK, rest)])

    @pl.when(s == 0)
    def _():
        pltpu.sync_copy(sums_hbm, sums_sh)

    plsc.subcore_barrier()

    def issue_loads(k, par):
        e0 = e_base + k * CHUNK
        pltpu.async_copy(rad_hbm.at[pl.ds(e0, CHUNK)], bufs[par],
                         lsems[par])
        pltpu.async_copy(snd_hbm.at[pl.ds(e0, CHUNK)], snds[par], nsems[par])

    def wait_radial(k, par):
        e0 = e_base + k * CHUNK
        pltpu.make_async_copy(rad_hbm.at[pl.ds(e0, CHUNK)], bufs[par],
                              lsems[par]).wait()

    def wait_snd(k, par):
        e0 = e_base + k * CHUNK
        pltpu.make_async_copy(snd_hbm.at[pl.ds(e0, CHUNK)], snds[par],
                              nsems[par]).wait()

    def wait_rcv(k, par):
        e0 = e_base + k * CHUNK
        pltpu.make_async_copy(rcv_hbm.at[pl.ds(e0, CHUNK)], rcv_v,
                              rsems[par]).wait()

    def issue_sgather(par):
        pltpu.async_copy(sums_sh.at[snds[par]], svs[par], ssems[par])

    def wait_sgather(par):
        pltpu.make_async_copy(sums_sh.at[snds[par]], svs[par],
                              ssems[par]).wait()

    def wait_drain(par):
        pltpu.make_async_copy(bufs[par], cch_hbm.at[pl.ds(0, CHUNK)],
                              dsems[par]).wait()

    def process(k, par, issue_next, drain_guard):
        other = 1 - par
        if issue_next:
            if drain_guard is None:
                wait_drain(other)
            else:
                @pl.when(drain_guard)
                def _():
                    wait_drain(other)
            issue_loads(k + 1, other)
        wait_radial(k, par)
        wait_rcv(k, par)
        wait_sgather(par)
        buf = bufs[par]
        sv = svs[par]

        def grp(g, gc):
            b = g * 16
            s16 = sv[pl.ds(b, 16)]
            rows = lanes + b
            for j in range(PAYLOAD):
                cols = jnp.full((16,), j, jnp.int32)
                v = plsc.load_gather(buf, [rows, cols])
                plsc.store_scatter(buf, [rows, cols], v * s16)
            return gc

        lax.fori_loop(0, GROUPS, grp, 0)
        e0 = e_base + k * CHUNK
        pltpu.async_copy(buf, cch_hbm.at[pl.ds(e0, CHUNK)], dsems[par])
        pltpu.sync_copy(buf, a_sh.at[rcv_v], add=True)
        if issue_next:
            e1 = e_base + (k + 1) * CHUNK
            pltpu.async_copy(rcv_hbm.at[pl.ds(e1, CHUNK)], rcv_v,
                             rsems[other])
            wait_snd(k + 1, other)
            issue_sgather(other)

    issue_loads(0, 0)
    pltpu.async_copy(rcv_hbm.at[pl.ds(e_base, CHUNK)], rcv_v, rsems[0])
    wait_snd(0, 0)
    issue_sgather(0)

    def outer(i, carry):
        k0 = 2 * i
        process(k0, 0, True, i > 0)
        process(k0 + 1, 1, True, None)
        return carry

    lax.fori_loop(0, (CHUNKS_PER_TILE - 1) // 2, outer, 0)
    process(CHUNKS_PER_TILE - 1, 0, False, None)
    wait_drain(0)
    wait_drain(1)
    plsc.subcore_barrier()
    pltpu.sync_copy(a_sh.at[pl.ds(rz, ROWS_PER_TILE)],
                    ach_hbm.at[pl.ds(rz, ROWS_PER_TILE)])


def _sc_run(sums, sender, rcv, radial):
    run = pl.kernel(
        _sc_body,
        out_type=[
            jax.ShapeDtypeStruct((2, N_A, PAYLOAD), jnp.float32),
            jax.ShapeDtypeStruct((2, N_EDGES, PAYLOAD), jnp.float32),
        ],
        mesh=plsc.VectorSubcoreMesh(core_axis_name="c", subcore_axis_name="s",
                                    num_cores=2),
        scratch_types=[
            pltpu.VMEM((CHUNK, PAYLOAD), jnp.float32),
            pltpu.VMEM((CHUNK, PAYLOAD), jnp.float32),
            pltpu.VMEM((CHUNK,), jnp.int32),
            pltpu.VMEM((CHUNK,), jnp.int32),
            pltpu.VMEM((CHUNK,), jnp.int32),
            pltpu.VMEM((CHUNK,), jnp.float32),
            pltpu.VMEM((CHUNK,), jnp.float32),
            pltpu.VMEM_SHARED((N_NODES,), jnp.float32),
            pltpu.VMEM_SHARED((N_A, PAYLOAD), jnp.float32),
            pltpu.SemaphoreType.DMA,
            pltpu.SemaphoreType.DMA,
            pltpu.SemaphoreType.DMA,
            pltpu.SemaphoreType.DMA,
            pltpu.SemaphoreType.DMA,
            pltpu.SemaphoreType.DMA,
            pltpu.SemaphoreType.DMA,
            pltpu.SemaphoreType.DMA,
            pltpu.SemaphoreType.DMA,
            pltpu.SemaphoreType.DMA,
        ],
        compiler_params=pltpu.CompilerParams(
            needs_layout_passes=False, use_tc_tiling_on_sc=False),
    )
    return run(sums, sender, rcv, radial)


def kernel(edge_index, radial_feature, node_feats):
    e = radial_feature.shape[1]
    k, i, j = radial_feature.shape[2:]
    n = node_feats.shape[0]
    sender = edge_index[0].astype(jnp.int32)
    receiver = edge_index[1].astype(jnp.int32)
    radial = radial_feature.reshape(2, e, k * i * j)
    sums = _node_sums(node_feats)
    a, comb = _sc_run(sums, sender, receiver, radial)
    return (
        a[0, :n].reshape(n, k, i, j),
        a[1, :n].reshape(n, k, i, j),
        comb[0].reshape(e, k, i, j),
        comb[1].reshape(e, k, i, j),
    )


# speedup vs baseline: 1.1794x; 1.0031x over previous
import jax
import jax.numpy as jnp
from jax import lax
from jax.experimental import pallas as pl
from jax.experimental.pallas import tpu as pltpu
from jax.experimental.pallas import tpu_sc as plsc

N_NODES = 50000
N_EDGES = 800000
PAYLOAD = 32

NUM_SUBCORES = 16
EDGES_PER_TILE = N_EDGES // NUM_SUBCORES
CHUNK = 400
CHUNKS_PER_TILE = EDGES_PER_TILE // CHUNK
GROUPS = CHUNK // 16
SCATTER_ROWS = 100
SCATTERS = CHUNK // SCATTER_ROWS
N_A = 50048
ROWS_PER_TILE = N_A // NUM_SUBCORES
ZCOPIES = ROWS_PER_TILE // CHUNK


def _sums_body(x_ref, o_ref):
    o_ref[...] = jnp.sum(x_ref[...], axis=1, keepdims=True)


def _node_sums(node_feats):
    n = node_feats.shape[0]
    out = pl.pallas_call(
        _sums_body,
        grid=(50,),
        in_specs=[pl.BlockSpec((n // 50, 128), lambda i: (i, 0))],
        out_specs=pl.BlockSpec((n // 50, 1), lambda i: (i, 0)),
        out_shape=jax.ShapeDtypeStruct((n, 1), jnp.float32),
    )(node_feats)
    return out.reshape(n)


def _sc_body(sums_hbm, snd_hbm, rcv_hbm, radial_hbm, a_hbm, comb_hbm,
             buf0, buf1, snd0, snd1, rcv0, rcv1, s0, s1, a_sh,
             lsem0, lsem1, dsem0, dsem1, ssem0, ssem1,
             nsem0, nsem1, rsem0, rsem1, asem0, asem1):
    s = lax.axis_index("s")
    c = lax.axis_index("c")
    zeros16 = jnp.zeros((16,), jnp.float32)
    lanes = lax.iota(jnp.int32, 16)
    e_base = s * EDGES_PER_TILE
    rz = s * ROWS_PER_TILE
    rad_hbm = radial_hbm.at[c]
    ach_hbm = a_hbm.at[c]
    cch_hbm = comb_hbm.at[c]
    bufs, snds, svs = (buf0, buf1), (snd0, snd1), (s0, s1)
    rcvs = (rcv0, rcv1)
    lsems, dsems, ssems = (lsem0, lsem1), (dsem0, dsem1), (ssem0, ssem1)
    nsems, rsems, asems = (nsem0, nsem1), (rsem0, rsem1), (asem0, asem1)

    def zrow(i, carry):
        buf0[i, pl.ds(0, 16)] = zeros16
        buf0[i, pl.ds(16, 16)] = zeros16
        return carry

    lax.fori_loop(0, CHUNK, zrow, 0)
    for i in range(ZCOPIES):
        pltpu.sync_copy(buf0.at[pl.ds(0, CHUNK)],
                        a_sh.at[pl.ds(rz + i * CHUNK, CHUNK)])
    rest = ROWS_PER_TILE - ZCOPIES * CHUNK
    pltpu.sync_copy(buf0.at[pl.ds(0, rest)],
                    a_sh.at[pl.ds(rz + ZCOPIES * CHUNK, rest)])

    plsc.subcore_barrier()

    def issue_loads(k, par):
        e0 = e_base + k * CHUNK
        pltpu.async_copy(rad_hbm.at[pl.ds(e0, CHUNK)], bufs[par],
                         lsems[par])
        pltpu.async_copy(snd_hbm.at[pl.ds(e0, CHUNK)], snds[par], nsems[par])
        pltpu.async_copy(rcv_hbm.at[pl.ds(e0, CHUNK)], rcvs[par], rsems[par])

    def wait_radial(k, par):
        e0 = e_base + k * CHUNK
        pltpu.make_async_copy(rad_hbm.at[pl.ds(e0, CHUNK)], bufs[par],
                              lsems[par]).wait()

    def wait_snd(k, par):
        e0 = e_base + k * CHUNK
        pltpu.make_async_copy(snd_hbm.at[pl.ds(e0, CHUNK)], snds[par],
                              nsems[par]).wait()

    def wait_rcv(k, par):
        e0 = e_base + k * CHUNK
        pltpu.make_async_copy(rcv_hbm.at[pl.ds(e0, CHUNK)], rcvs[par],
                              rsems[par]).wait()

    def issue_sgather(par):
        pltpu.async_copy(sums_hbm.at[snds[par]], svs[par], ssems[par])

    def wait_sgather(par):
        pltpu.make_async_copy(sums_hbm.at[snds[par]], svs[par],
                              ssems[par]).wait()

    def wait_drain(par):
        pltpu.make_async_copy(bufs[par], cch_hbm.at[pl.ds(0, CHUNK)],
                              dsems[par]).wait()

    def wait_scatter(par):
        pltpu.make_async_copy(bufs[par], a_sh.at[rcvs[par]],
                              asems[par]).wait()

    def process(k, par, issue_next, drain_guard):
        other = 1 - par
        if issue_next:
            if drain_guard is None:
                wait_drain(other)
                wait_scatter(other)
            else:
                @pl.when(drain_guard)
                def _():
                    wait_drain(other)
                    wait_scatter(other)
            issue_loads(k + 1, other)
        wait_radial(k, par)
        wait_rcv(k, par)
        wait_sgather(par)
        buf = bufs[par]
        sv = svs[par]

        def grp(g, gc):
            b = g * 16
            s16 = sv[pl.ds(b, 16)]
            rows = lanes + b
            for j in range(PAYLOAD):
                cols = jnp.full((16,), j, jnp.int32)
                v = plsc.load_gather(buf, [rows, cols])
                plsc.store_scatter(buf, [rows, cols], v * s16)
            return gc

        lax.fori_loop(0, GROUPS, grp, 0)
        e0 = e_base + k * CHUNK
        pltpu.async_copy(buf, cch_hbm.at[pl.ds(e0, CHUNK)], dsems[par])
        pltpu.async_copy(buf, a_sh.at[rcvs[par]], asems[par], add=True)
        if issue_next:
            wait_snd(k + 1, other)
            issue_sgather(other)

    issue_loads(0, 0)
    wait_snd(0, 0)
    issue_sgather(0)

    def outer(i, carry):
        k0 = 2 * i
        process(k0, 0, True, i > 0)
        process(k0 + 1, 1, True, None)
        return carry

    lax.fori_loop(0, (CHUNKS_PER_TILE - 1) // 2, outer, 0)
    process(CHUNKS_PER_TILE - 1, 0, False, None)
    wait_drain(0)
    wait_drain(1)
    wait_scatter(0)
    wait_scatter(1)
    plsc.subcore_barrier()
    pltpu.sync_copy(a_sh.at[pl.ds(rz, ROWS_PER_TILE)],
                    ach_hbm.at[pl.ds(rz, ROWS_PER_TILE)])


def _sc_run(sums, sender, rcv, radial):
    run = pl.kernel(
        _sc_body,
        out_type=[
            jax.ShapeDtypeStruct((2, N_A, PAYLOAD), jnp.float32),
            jax.ShapeDtypeStruct((2, N_EDGES, PAYLOAD), jnp.float32),
        ],
        mesh=plsc.VectorSubcoreMesh(core_axis_name="c", subcore_axis_name="s",
                                    num_cores=2),
        scratch_types=[
            pltpu.VMEM((CHUNK, PAYLOAD), jnp.float32),
            pltpu.VMEM((CHUNK, PAYLOAD), jnp.float32),
            pltpu.VMEM((CHUNK,), jnp.int32),
            pltpu.VMEM((CHUNK,), jnp.int32),
            pltpu.VMEM((CHUNK,), jnp.int32),
            pltpu.VMEM((CHUNK,), jnp.int32),
            pltpu.VMEM((CHUNK,), jnp.float32),
            pltpu.VMEM((CHUNK,), jnp.float32),
            pltpu.VMEM_SHARED((N_A, PAYLOAD), jnp.float32),
            pltpu.SemaphoreType.DMA,
            pltpu.SemaphoreType.DMA,
            pltpu.SemaphoreType.DMA,
            pltpu.SemaphoreType.DMA,
            pltpu.SemaphoreType.DMA,
            pltpu.SemaphoreType.DMA,
            pltpu.SemaphoreType.DMA,
            pltpu.SemaphoreType.DMA,
            pltpu.SemaphoreType.DMA,
            pltpu.SemaphoreType.DMA,
            pltpu.SemaphoreType.DMA,
            pltpu.SemaphoreType.DMA,
        ],
        compiler_params=pltpu.CompilerParams(
            needs_layout_passes=False, use_tc_tiling_on_sc=False),
    )
    return run(sums, sender, rcv, radial)


def kernel(edge_index, radial_feature, node_feats):
    e = radial_feature.shape[1]
    k, i, j = radial_feature.shape[2:]
    n = node_feats.shape[0]
    sender = edge_index[0].astype(jnp.int32)
    receiver = edge_index[1].astype(jnp.int32)
    radial = radial_feature.reshape(2, e, k * i * j)
    sums = _node_sums(node_feats)
    a, comb = _sc_run(sums, sender, receiver, radial)
    return (
        a[0, :n].reshape(n, k, i, j),
        a[1, :n].reshape(n, k, i, j),
        comb[0].reshape(e, k, i, j),
        comb[1].reshape(e, k, i, j),
    )
